# baseline (speedup 1.0000x reference)
import functools

import jax
import jax.numpy as jnp
from jax import lax
from jax.experimental import pallas as pl
from jax.experimental.pallas import tpu as pltpu
from jax.experimental.pallas import tpu_sc as plsc

N = 10000
E = 320000
D = 128
L = 3
K = 32

NC = 2
NS = 16
NW = NC * NS
EPW = E // NW
CHUNK = 80
NCHUNK = EPW // CHUNK
NRING = 2
NTAIL = NCHUNK - (NCHUNK // NRING) * NRING
RPW = 624
ZR = 16

BR = 80


def _maxk_mask(t, _roll=None):
    if _roll is None:
        _roll = lambda v, s: pltpu.roll(v, s, 1)
    n = t.shape[1]
    cols = lax.broadcasted_iota(jnp.int32, t.shape, 1)
    u = lax.bitcast_convert_type(t, jnp.int32)
    m = u ^ (lax.shift_right_arithmetic(u, 31) & jnp.int32(0x7FFFFFFF))
    key0 = (m & jnp.int32(-128)) | (127 - cols)
    x = key0
    k = 2
    while k <= n // 2:
        kbit0 = (cols & k) == 0
        j = k // 2
        while j >= 1:
            jbit0 = (cols & j) == 0
            pv = jnp.where(jbit0, _roll(x, n - j), _roll(x, j))
            take_min = kbit0 == jbit0
            x = jnp.where(take_min, jnp.minimum(x, pv), jnp.maximum(x, pv))
            j //= 2
        k *= 2
    for j in (n // 2, n // 4):
        jbit0 = (cols & j) == 0
        pv = jnp.where(jbit0, _roll(x, n - j), _roll(x, j))
        x = jnp.where(jbit0, jnp.minimum(x, pv), jnp.maximum(x, pv))
    m = jnp.where(cols >= n - K, x, jnp.int32(0x7FFFFFFF))
    sh = K // 2
    while sh >= 1:
        m = jnp.minimum(m, _roll(m, n - sh))
        sh //= 2
    thr = lax.slice(m, (0, n - K), (t.shape[0], n - K + 1))
    return jnp.where(key0 >= thr, t, 0.0)


def _tc_in_body(x_ref, wi_ref, bi_ref, w0_ref, b0_ref, o_ref):
    h0 = jnp.dot(x_ref[...], wi_ref[...], preferred_element_type=jnp.float32)
    h0 = jnp.maximum(h0 + bi_ref[...], 0.0)
    t = jnp.dot(h0, w0_ref[...], preferred_element_type=jnp.float32) + b0_ref[...]
    o_ref[...] = _maxk_mask(t)


def _tc_mid_body(hs_ref, p0_ref, p1_ref, e_ref, w_ref, b_ref, o_ref):
    h = e_ref[...] * hs_ref[...] + p0_ref[...] + p1_ref[...]
    t = jnp.dot(h, w_ref[...], preferred_element_type=jnp.float32) + b_ref[...]
    o_ref[...] = _maxk_mask(t)


def _tc_out_body(hs_ref, p0_ref, p1_ref, e_ref, w_ref, b_ref, o_ref):
    h = e_ref[...] * hs_ref[...] + p0_ref[...] + p1_ref[...]
    o_ref[...] = jnp.dot(h, w_ref[...], preferred_element_type=jnp.float32) + b_ref[...]


def _rows():
    return pl.BlockSpec((BR, D), lambda i: (i, 0))


def _mat():
    return pl.BlockSpec((D, D), lambda i: (0, 0))


def _vec():
    return pl.BlockSpec((1, D), lambda i: (0, 0))


_out_rows = jax.ShapeDtypeStruct((N, D), jnp.float32)

_tc_in = pl.pallas_call(
    _tc_in_body, grid=(N // BR,),
    in_specs=[_rows(), _mat(), _vec(), _mat(), _vec()],
    out_specs=_rows(), out_shape=_out_rows)

_tc_mid = pl.pallas_call(
    _tc_mid_body, grid=(N // BR,),
    in_specs=[_rows(), _rows(), _rows(), _vec(), _mat(), _vec()],
    out_specs=_rows(), out_shape=_out_rows)

_tc_out = pl.pallas_call(
    _tc_out_body, grid=(N // BR,),
    in_specs=[_rows(), _rows(), _rows(), _vec(), _mat(), _vec()],
    out_specs=_rows(), out_shape=_out_rows)


@functools.lru_cache(maxsize=None)
def _make_sc_agg():
    mesh = plsc.VectorSubcoreMesh(core_axis_name="c", subcore_axis_name="s")
    return pl.kernel(
        _sc_agg_body,
        mesh=mesh,
        out_type=jax.ShapeDtypeStruct((NC * N, D), jnp.float32),
        scratch_types=[pltpu.VMEM((CHUNK, D), jnp.float32) for _ in range(NRING)]
        + [pltpu.VMEM((CHUNK,), jnp.int32) for _ in range(2 * NRING)]
        + [pltpu.VMEM((ZR, D), jnp.float32)]
        + [pltpu.VMEM_SHARED((N, D), jnp.float32)]
        + [pltpu.SemaphoreType.DMA for _ in range(4 * NRING)],
    )


def _sc_agg_body(hs_hbm, src_hbm, dst_hbm, out_hbm, *rest):
    rows = rest[0:NRING]
    sbuf = rest[NRING:2 * NRING]
    dbuf = rest[2 * NRING:3 * NRING]
    zbuf = rest[3 * NRING]
    acc_sh = rest[3 * NRING + 1]
    sems = rest[3 * NRING + 2:]
    gs = sems[0:NRING]
    ss = sems[NRING:2 * NRING]
    isx = sems[2 * NRING:3 * NRING]
    jsx = sems[3 * NRING:4 * NRING]

    cid = lax.axis_index("c")
    sid = lax.axis_index("s")
    wg = cid * NS + sid

    def _zrow(r, carry):
        for c in range(D // 16):
            zbuf[r, pl.ds(c * 16, 16)] = jnp.zeros((16,), jnp.float32)
        return carry

    lax.fori_loop(0, ZR, _zrow, 0)

    base = pl.multiple_of(sid * RPW, 16)

    def _zcp(j, carry):
        pltpu.sync_copy(zbuf, acc_sh.at[pl.ds(base + j * ZR, ZR)])
        return carry

    lax.fori_loop(0, RPW // ZR, _zcp, 0)

    @pl.when(sid == NS - 1)
    def _zero_tail():
        pltpu.sync_copy(zbuf, acc_sh.at[pl.ds(NS * RPW, N - NS * RPW)])

    plsc.subcore_barrier()

    def fire_isrc(c, b):
        pltpu.async_copy(src_hbm.at[wg, c], sbuf[b], isx[b])

    def wait_isrc(b):
        pltpu.make_async_copy(src_hbm.at[wg, 0], sbuf[b], isx[b]).wait()

    def fire_idst(c, b):
        pltpu.async_copy(dst_hbm.at[wg, c], dbuf[b], jsx[b])

    def wait_idst(b):
        pltpu.make_async_copy(dst_hbm.at[wg, 0], dbuf[b], jsx[b]).wait()

    def fire_gather(b):
        pltpu.async_copy(hs_hbm.at[sbuf[b]], rows[b], gs[b])

    def wait_gather(b):
        pltpu.make_async_copy(hs_hbm.at[sbuf[b]], rows[b], gs[b]).wait()

    def fire_scatter(b):
        pltpu.async_copy(rows[b], acc_sh.at[dbuf[b]], ss[b], add=True)

    def wait_scatter(b):
        pltpu.make_async_copy(rows[b], acc_sh.at[dbuf[b]], ss[b]).wait()

    for b in range(NRING):
        fire_isrc(b, b)
        fire_idst(b, b)
    for b in range(NRING):
        wait_isrc(b)
        fire_gather(b)

    def _grp(k, carry):
        for b in range(NRING):
            c = k * NRING + b
            wait_gather(b)
            wait_idst(b)
            fire_scatter(b)

            @pl.when(c + NRING < NCHUNK)
            def _pre(b=b, c=c):
                fire_isrc(c + NRING, b)

            wait_scatter(b)

            @pl.when(c + NRING < NCHUNK)
            def _nxt(b=b, c=c):
                fire_idst(c + NRING, b)
                wait_isrc(b)
                fire_gather(b)
        return carry

    lax.fori_loop(0, NCHUNK // NRING, _grp, 0)

    for b in range(NTAIL):
        wait_gather(b)
        wait_idst(b)
        fire_scatter(b)
        wait_scatter(b)

    plsc.subcore_barrier()
    obase = pl.multiple_of(cid * N + sid * RPW, 16)
    pltpu.sync_copy(acc_sh.at[pl.ds(base, RPW)], out_hbm.at[pl.ds(obase, RPW)])

    @pl.when(sid == NS - 1)
    def _write_tail():
        pltpu.sync_copy(acc_sh.at[pl.ds(NS * RPW, N - NS * RPW)],
                        out_hbm.at[pl.ds(cid * N + NS * RPW, N - NS * RPW)])


def kernel(x, edge_index, W_in, b_in, Ws, bs, eps, W_out, b_out):
    src = edge_index[0].astype(jnp.int32).reshape(NW, NCHUNK, CHUNK)
    dst = edge_index[1].astype(jnp.int32).reshape(NW, NCHUNK, CHUNK)
    ones = jnp.ones((1, D), jnp.float32)

    hs = _tc_in(x, W_in, b_in.reshape(1, D), Ws[0], bs[0].reshape(1, D))
    for i in range(L):
        part = _make_sc_agg()(hs, src, dst)
        p0, p1 = part[:N], part[N:]
        epsb = (1.0 + eps[i]) * ones
        if i < L - 1:
            hs = _tc_mid(hs, p0, p1, epsb, Ws[i + 1], bs[i + 1].reshape(1, D))
        else:
            return _tc_out(hs, p0, p1, epsb, W_out, b_out.reshape(1, D))


# speedup vs baseline: 2.1089x; 2.1089x over previous
import functools

import jax
import jax.numpy as jnp
from jax import lax
from jax.experimental import pallas as pl
from jax.experimental.pallas import tpu as pltpu
from jax.experimental.pallas import tpu_sc as plsc

N = 10000
E = 320000
D = 128
L = 3
K = 32

NC = 2
NS = 16
NW = NC * NS
EPW = E // NW
CHUNK = 80
NCHUNK = EPW // CHUNK
NRING = 2
NTAIL = NCHUNK - (NCHUNK // NRING) * NRING
RPW = 624
ZR = 16

BR = 2000


def _maxk_mask(t, _roll=None):
    if _roll is None:
        _roll = lambda v, s: pltpu.roll(v, s, 1)
    n = t.shape[1]
    cols = lax.broadcasted_iota(jnp.int32, t.shape, 1)
    u = lax.bitcast_convert_type(t, jnp.int32)
    m = u ^ (lax.shift_right_arithmetic(u, 31) & jnp.int32(0x7FFFFFFF))
    key0 = (m & jnp.int32(-128)) | (127 - cols)
    x = key0
    k = 2
    while k <= n // 2:
        kbit0 = (cols & k) == 0
        j = k // 2
        while j >= 1:
            jbit0 = (cols & j) == 0
            pv = jnp.where(jbit0, _roll(x, n - j), _roll(x, j))
            take_min = kbit0 == jbit0
            x = jnp.where(take_min, jnp.minimum(x, pv), jnp.maximum(x, pv))
            j //= 2
        k *= 2
    for j in (n // 2, n // 4):
        jbit0 = (cols & j) == 0
        pv = jnp.where(jbit0, _roll(x, n - j), _roll(x, j))
        x = jnp.where(jbit0, jnp.minimum(x, pv), jnp.maximum(x, pv))
    m = jnp.where(cols >= n - K, x, jnp.int32(0x7FFFFFFF))
    sh = K // 2
    while sh >= 1:
        m = jnp.minimum(m, _roll(m, n - sh))
        sh //= 2
    thr = lax.slice(m, (0, n - K), (t.shape[0], n - K + 1))
    return jnp.where(key0 >= thr, t, 0.0)


def _tc_in_body(x_ref, wi_ref, bi_ref, w0_ref, b0_ref, o_ref):
    h0 = jnp.dot(x_ref[...], wi_ref[...], preferred_element_type=jnp.float32)
    h0 = jnp.maximum(h0 + bi_ref[...], 0.0)
    t = jnp.dot(h0, w0_ref[...], preferred_element_type=jnp.float32) + b0_ref[...]
    o_ref[...] = _maxk_mask(t)


def _tc_mid_body(hs_ref, p0_ref, p1_ref, e_ref, w_ref, b_ref, o_ref):
    h = e_ref[...] * hs_ref[...] + p0_ref[...] + p1_ref[...]
    t = jnp.dot(h, w_ref[...], preferred_element_type=jnp.float32) + b_ref[...]
    o_ref[...] = _maxk_mask(t)


def _tc_out_body(hs_ref, p0_ref, p1_ref, e_ref, w_ref, b_ref, o_ref):
    h = e_ref[...] * hs_ref[...] + p0_ref[...] + p1_ref[...]
    o_ref[...] = jnp.dot(h, w_ref[...], preferred_element_type=jnp.float32) + b_ref[...]


def _rows():
    return pl.BlockSpec((BR, D), lambda i: (i, 0))


def _mat():
    return pl.BlockSpec((D, D), lambda i: (0, 0))


def _vec():
    return pl.BlockSpec((1, D), lambda i: (0, 0))


_out_rows = jax.ShapeDtypeStruct((N, D), jnp.float32)

_tc_in = pl.pallas_call(
    _tc_in_body, grid=(N // BR,),
    in_specs=[_rows(), _mat(), _vec(), _mat(), _vec()],
    out_specs=_rows(), out_shape=_out_rows)

_tc_mid = pl.pallas_call(
    _tc_mid_body, grid=(N // BR,),
    in_specs=[_rows(), _rows(), _rows(), _vec(), _mat(), _vec()],
    out_specs=_rows(), out_shape=_out_rows)

_tc_out = pl.pallas_call(
    _tc_out_body, grid=(N // BR,),
    in_specs=[_rows(), _rows(), _rows(), _vec(), _mat(), _vec()],
    out_specs=_rows(), out_shape=_out_rows)


@functools.lru_cache(maxsize=None)
def _make_sc_agg():
    mesh = plsc.VectorSubcoreMesh(core_axis_name="c", subcore_axis_name="s")
    return pl.kernel(
        _sc_agg_body,
        mesh=mesh,
        out_type=jax.ShapeDtypeStruct((NC * N, D), jnp.float32),
        scratch_types=[pltpu.VMEM((CHUNK, D), jnp.float32) for _ in range(NRING)]
        + [pltpu.VMEM((CHUNK,), jnp.int32) for _ in range(2 * NRING)]
        + [pltpu.VMEM((ZR, D), jnp.float32)]
        + [pltpu.VMEM_SHARED((N, D), jnp.float32)]
        + [pltpu.SemaphoreType.DMA for _ in range(4 * NRING)],
    )


def _sc_agg_body(hs_hbm, src_hbm, dst_hbm, out_hbm, *rest):
    rows = rest[0:NRING]
    sbuf = rest[NRING:2 * NRING]
    dbuf = rest[2 * NRING:3 * NRING]
    zbuf = rest[3 * NRING]
    acc_sh = rest[3 * NRING + 1]
    sems = rest[3 * NRING + 2:]
    gs = sems[0:NRING]
    ss = sems[NRING:2 * NRING]
    isx = sems[2 * NRING:3 * NRING]
    jsx = sems[3 * NRING:4 * NRING]

    cid = lax.axis_index("c")
    sid = lax.axis_index("s")
    wg = cid * NS + sid

    def _zrow(r, carry):
        for c in range(D // 16):
            zbuf[r, pl.ds(c * 16, 16)] = jnp.zeros((16,), jnp.float32)
        return carry

    lax.fori_loop(0, ZR, _zrow, 0)

    base = pl.multiple_of(sid * RPW, 16)

    def _zcp(j, carry):
        pltpu.sync_copy(zbuf, acc_sh.at[pl.ds(base + j * ZR, ZR)])
        return carry

    lax.fori_loop(0, RPW // ZR, _zcp, 0)

    @pl.when(sid == NS - 1)
    def _zero_tail():
        pltpu.sync_copy(zbuf, acc_sh.at[pl.ds(NS * RPW, N - NS * RPW)])

    plsc.subcore_barrier()

    def fire_isrc(c, b):
        pltpu.async_copy(src_hbm.at[wg, c], sbuf[b], isx[b])

    def wait_isrc(b):
        pltpu.make_async_copy(src_hbm.at[wg, 0], sbuf[b], isx[b]).wait()

    def fire_idst(c, b):
        pltpu.async_copy(dst_hbm.at[wg, c], dbuf[b], jsx[b])

    def wait_idst(b):
        pltpu.make_async_copy(dst_hbm.at[wg, 0], dbuf[b], jsx[b]).wait()

    def fire_gather(b):
        pltpu.async_copy(hs_hbm.at[sbuf[b]], rows[b], gs[b])

    def wait_gather(b):
        pltpu.make_async_copy(hs_hbm.at[sbuf[b]], rows[b], gs[b]).wait()

    def fire_scatter(b):
        pltpu.async_copy(rows[b], acc_sh.at[dbuf[b]], ss[b], add=True)

    def wait_scatter(b):
        pltpu.make_async_copy(rows[b], acc_sh.at[dbuf[b]], ss[b]).wait()

    for b in range(NRING):
        fire_isrc(b, b)
        fire_idst(b, b)
    for b in range(NRING):
        wait_isrc(b)
        fire_gather(b)

    def _grp(k, carry):
        for b in range(NRING):
            c = k * NRING + b
            wait_gather(b)
            wait_idst(b)
            fire_scatter(b)

            @pl.when(c + NRING < NCHUNK)
            def _pre(b=b, c=c):
                fire_isrc(c + NRING, b)

            wait_scatter(b)

            @pl.when(c + NRING < NCHUNK)
            def _nxt(b=b, c=c):
                fire_idst(c + NRING, b)
                wait_isrc(b)
                fire_gather(b)
        return carry

    lax.fori_loop(0, NCHUNK // NRING, _grp, 0)

    for b in range(NTAIL):
        wait_gather(b)
        wait_idst(b)
        fire_scatter(b)
        wait_scatter(b)

    plsc.subcore_barrier()
    obase = pl.multiple_of(cid * N + sid * RPW, 16)
    pltpu.sync_copy(acc_sh.at[pl.ds(base, RPW)], out_hbm.at[pl.ds(obase, RPW)])

    @pl.when(sid == NS - 1)
    def _write_tail():
        pltpu.sync_copy(acc_sh.at[pl.ds(NS * RPW, N - NS * RPW)],
                        out_hbm.at[pl.ds(cid * N + NS * RPW, N - NS * RPW)])


def kernel(x, edge_index, W_in, b_in, Ws, bs, eps, W_out, b_out):
    src = edge_index[0].astype(jnp.int32).reshape(NW, NCHUNK, CHUNK)
    dst = edge_index[1].astype(jnp.int32).reshape(NW, NCHUNK, CHUNK)
    ones = jnp.ones((1, D), jnp.float32)

    hs = _tc_in(x, W_in, b_in.reshape(1, D), Ws[0], bs[0].reshape(1, D))
    for i in range(L):
        part = _make_sc_agg()(hs, src, dst)
        p0, p1 = part[:N], part[N:]
        epsb = (1.0 + eps[i]) * ones
        if i < L - 1:
            hs = _tc_mid(hs, p0, p1, epsb, Ws[i + 1], bs[i + 1].reshape(1, D))
        else:
            return _tc_out(hs, p0, p1, epsb, W_out, b_out.reshape(1, D))


# speedup vs baseline: 2.3682x; 1.1229x over previous
import functools

import jax
import jax.numpy as jnp
from jax import lax
from jax.experimental import pallas as pl
from jax.experimental.pallas import tpu as pltpu
from jax.experimental.pallas import tpu_sc as plsc

N = 10000
E = 320000
D = 128
L = 3
K = 32

NC = 2
NS = 16
NW = NC * NS
EPW = E // NW
CHUNK = 80
NCHUNK = EPW // CHUNK
NRING = 3
NTAIL = NCHUNK - (NCHUNK // NRING) * NRING
RPW = 624
ZR = 16

BR = 2000


def _maxk_mask(t, _roll=None):
    if _roll is None:
        _roll = lambda v, s: pltpu.roll(v, s, 1)
    n = t.shape[1]
    cols = lax.broadcasted_iota(jnp.int32, t.shape, 1)
    u = lax.bitcast_convert_type(t, jnp.int32)
    m = u ^ (lax.shift_right_arithmetic(u, 31) & jnp.int32(0x7FFFFFFF))
    key0 = (m & jnp.int32(-128)) | (127 - cols)
    x = key0
    k = 2
    while k <= n // 2:
        kbit0 = (cols & k) == 0
        j = k // 2
        while j >= 1:
            jbit0 = (cols & j) == 0
            pv = jnp.where(jbit0, _roll(x, n - j), _roll(x, j))
            take_min = kbit0 == jbit0
            x = jnp.where(take_min, jnp.minimum(x, pv), jnp.maximum(x, pv))
            j //= 2
        k *= 2
    for j in (n // 2, n // 4):
        jbit0 = (cols & j) == 0
        pv = jnp.where(jbit0, _roll(x, n - j), _roll(x, j))
        x = jnp.where(jbit0, jnp.minimum(x, pv), jnp.maximum(x, pv))
    m = jnp.where(cols >= n - K, x, jnp.int32(0x7FFFFFFF))
    sh = K // 2
    while sh >= 1:
        m = jnp.minimum(m, _roll(m, n - sh))
        sh //= 2
    thr = lax.slice(m, (0, n - K), (t.shape[0], n - K + 1))
    return jnp.where(key0 >= thr, t, 0.0)


def _tc_in_body(x_ref, wi_ref, bi_ref, w0_ref, b0_ref, o_ref):
    h0 = jnp.dot(x_ref[...], wi_ref[...], preferred_element_type=jnp.float32)
    h0 = jnp.maximum(h0 + bi_ref[...], 0.0)
    t = jnp.dot(h0, w0_ref[...], preferred_element_type=jnp.float32) + b0_ref[...]
    o_ref[...] = _maxk_mask(t)


def _tc_mid_body(hs_ref, p0_ref, p1_ref, e_ref, w_ref, b_ref, o_ref):
    h = e_ref[...] * hs_ref[...] + p0_ref[...] + p1_ref[...]
    t = jnp.dot(h, w_ref[...], preferred_element_type=jnp.float32) + b_ref[...]
    o_ref[...] = _maxk_mask(t)


def _tc_out_body(hs_ref, p0_ref, p1_ref, e_ref, w_ref, b_ref, o_ref):
    h = e_ref[...] * hs_ref[...] + p0_ref[...] + p1_ref[...]
    o_ref[...] = jnp.dot(h, w_ref[...], preferred_element_type=jnp.float32) + b_ref[...]


def _rows():
    return pl.BlockSpec((BR, D), lambda i: (i, 0))


def _mat():
    return pl.BlockSpec((D, D), lambda i: (0, 0))


def _vec():
    return pl.BlockSpec((1, D), lambda i: (0, 0))


_out_rows = jax.ShapeDtypeStruct((N, D), jnp.float32)

_tc_in = pl.pallas_call(
    _tc_in_body, grid=(N // BR,),
    in_specs=[_rows(), _mat(), _vec(), _mat(), _vec()],
    out_specs=_rows(), out_shape=_out_rows)

_tc_mid = pl.pallas_call(
    _tc_mid_body, grid=(N // BR,),
    in_specs=[_rows(), _rows(), _rows(), _vec(), _mat(), _vec()],
    out_specs=_rows(), out_shape=_out_rows)

_tc_out = pl.pallas_call(
    _tc_out_body, grid=(N // BR,),
    in_specs=[_rows(), _rows(), _rows(), _vec(), _mat(), _vec()],
    out_specs=_rows(), out_shape=_out_rows)


@functools.lru_cache(maxsize=None)
def _make_sc_agg():
    mesh = plsc.VectorSubcoreMesh(core_axis_name="c", subcore_axis_name="s")
    return pl.kernel(
        _sc_agg_body,
        mesh=mesh,
        out_type=jax.ShapeDtypeStruct((NC * N, D), jnp.float32),
        scratch_types=[pltpu.VMEM((CHUNK, D), jnp.float32) for _ in range(NRING)]
        + [pltpu.VMEM((CHUNK,), jnp.int32) for _ in range(2 * NRING)]
        + [pltpu.VMEM((ZR, D), jnp.float32)]
        + [pltpu.VMEM_SHARED((N, D), jnp.float32)]
        + [pltpu.SemaphoreType.DMA for _ in range(4 * NRING)],
    )


def _sc_agg_body(hs_hbm, src_hbm, dst_hbm, out_hbm, *rest):
    rows = rest[0:NRING]
    sbuf = rest[NRING:2 * NRING]
    dbuf = rest[2 * NRING:3 * NRING]
    zbuf = rest[3 * NRING]
    acc_sh = rest[3 * NRING + 1]
    sems = rest[3 * NRING + 2:]
    gs = sems[0:NRING]
    ss = sems[NRING:2 * NRING]
    isx = sems[2 * NRING:3 * NRING]
    jsx = sems[3 * NRING:4 * NRING]

    cid = lax.axis_index("c")
    sid = lax.axis_index("s")
    wg = cid * NS + sid

    def _zrow(r, carry):
        for c in range(D // 16):
            zbuf[r, pl.ds(c * 16, 16)] = jnp.zeros((16,), jnp.float32)
        return carry

    lax.fori_loop(0, ZR, _zrow, 0)

    base = pl.multiple_of(sid * RPW, 16)

    def _zcp(j, carry):
        pltpu.sync_copy(zbuf, acc_sh.at[pl.ds(base + j * ZR, ZR)])
        return carry

    lax.fori_loop(0, RPW // ZR, _zcp, 0)

    @pl.when(sid == NS - 1)
    def _zero_tail():
        pltpu.sync_copy(zbuf, acc_sh.at[pl.ds(NS * RPW, N - NS * RPW)])

    plsc.subcore_barrier()

    def fire_isrc(c, b):
        pltpu.async_copy(src_hbm.at[wg, c], sbuf[b], isx[b])

    def wait_isrc(b):
        pltpu.make_async_copy(src_hbm.at[wg, 0], sbuf[b], isx[b]).wait()

    def fire_idst(c, b):
        pltpu.async_copy(dst_hbm.at[wg, c], dbuf[b], jsx[b])

    def wait_idst(b):
        pltpu.make_async_copy(dst_hbm.at[wg, 0], dbuf[b], jsx[b]).wait()

    def fire_gather(b):
        pltpu.async_copy(hs_hbm.at[sbuf[b]], rows[b], gs[b])

    def wait_gather(b):
        pltpu.make_async_copy(hs_hbm.at[sbuf[b]], rows[b], gs[b]).wait()

    def fire_scatter(b):
        pltpu.async_copy(rows[b], acc_sh.at[dbuf[b]], ss[b], add=True)

    def wait_scatter(b):
        pltpu.make_async_copy(rows[b], acc_sh.at[dbuf[b]], ss[b]).wait()

    for b in range(NRING):
        fire_isrc(b, b)
        fire_idst(b, b)
    for b in range(NRING):
        wait_isrc(b)
        fire_gather(b)

    def _grp(k, carry):
        for b in range(NRING):
            c = k * NRING + b
            wait_gather(b)
            wait_idst(b)
            fire_scatter(b)

            @pl.when(c + NRING < NCHUNK)
            def _pre(b=b, c=c):
                fire_isrc(c + NRING, b)

            wait_scatter(b)

            @pl.when(c + NRING < NCHUNK)
            def _nxt(b=b, c=c):
                fire_idst(c + NRING, b)
                wait_isrc(b)
                fire_gather(b)
        return carry

    lax.fori_loop(0, NCHUNK // NRING, _grp, 0)

    for b in range(NTAIL):
        wait_gather(b)
        wait_idst(b)
        fire_scatter(b)
        wait_scatter(b)

    plsc.subcore_barrier()
    obase = pl.multiple_of(cid * N + sid * RPW, 16)
    pltpu.sync_copy(acc_sh.at[pl.ds(base, RPW)], out_hbm.at[pl.ds(obase, RPW)])

    @pl.when(sid == NS - 1)
    def _write_tail():
        pltpu.sync_copy(acc_sh.at[pl.ds(NS * RPW, N - NS * RPW)],
                        out_hbm.at[pl.ds(cid * N + NS * RPW, N - NS * RPW)])


def kernel(x, edge_index, W_in, b_in, Ws, bs, eps, W_out, b_out):
    src = edge_index[0].astype(jnp.int32).reshape(NW, NCHUNK, CHUNK)
    dst = edge_index[1].astype(jnp.int32).reshape(NW, NCHUNK, CHUNK)
    ones = jnp.ones((1, D), jnp.float32)

    hs = _tc_in(x, W_in, b_in.reshape(1, D), Ws[0], bs[0].reshape(1, D))
    for i in range(L):
        part = _make_sc_agg()(hs, src, dst)
        p0, p1 = part[:N], part[N:]
        epsb = (1.0 + eps[i]) * ones
        if i < L - 1:
            hs = _tc_mid(hs, p0, p1, epsb, Ws[i + 1], bs[i + 1].reshape(1, D))
        else:
            return _tc_out(hs, p0, p1, epsb, W_out, b_out.reshape(1, D))


# speedup vs baseline: 2.4207x; 1.0222x over previous
import functools

import jax
import jax.numpy as jnp
from jax import lax
from jax.experimental import pallas as pl
from jax.experimental.pallas import tpu as pltpu
from jax.experimental.pallas import tpu_sc as plsc

N = 10000
E = 320000
D = 128
L = 3
K = 32

NC = 2
NS = 16
NW = NC * NS
EPW = E // NW
CHUNK = 80
NCHUNK = EPW // CHUNK
NRING = 4
NTAIL = NCHUNK - (NCHUNK // NRING) * NRING
RPW = 624
ZR = 16

BR = 2000


def _maxk_mask(t, _roll=None):
    if _roll is None:
        _roll = lambda v, s: pltpu.roll(v, s, 1)
    n = t.shape[1]
    cols = lax.broadcasted_iota(jnp.int32, t.shape, 1)
    u = lax.bitcast_convert_type(t, jnp.int32)
    m = u ^ (lax.shift_right_arithmetic(u, 31) & jnp.int32(0x7FFFFFFF))
    key0 = (m & jnp.int32(-128)) | (127 - cols)
    x = key0
    k = 2
    while k <= n // 2:
        kbit0 = (cols & k) == 0
        j = k // 2
        while j >= 1:
            jbit0 = (cols & j) == 0
            pv = jnp.where(jbit0, _roll(x, n - j), _roll(x, j))
            take_min = kbit0 == jbit0
            x = jnp.where(take_min, jnp.minimum(x, pv), jnp.maximum(x, pv))
            j //= 2
        k *= 2
    for j in (n // 2, n // 4):
        jbit0 = (cols & j) == 0
        pv = jnp.where(jbit0, _roll(x, n - j), _roll(x, j))
        x = jnp.where(jbit0, jnp.minimum(x, pv), jnp.maximum(x, pv))
    m = jnp.where(cols >= n - K, x, jnp.int32(0x7FFFFFFF))
    sh = K // 2
    while sh >= 1:
        m = jnp.minimum(m, _roll(m, n - sh))
        sh //= 2
    thr = lax.slice(m, (0, n - K), (t.shape[0], n - K + 1))
    return jnp.where(key0 >= thr, t, 0.0)


def _tc_in_body(x_ref, wi_ref, bi_ref, w0_ref, b0_ref, o_ref):
    h0 = jnp.dot(x_ref[...], wi_ref[...], preferred_element_type=jnp.float32)
    h0 = jnp.maximum(h0 + bi_ref[...], 0.0)
    t = jnp.dot(h0, w0_ref[...], preferred_element_type=jnp.float32) + b0_ref[...]
    o_ref[...] = _maxk_mask(t)


def _tc_mid_body(hs_ref, p0_ref, p1_ref, e_ref, w_ref, b_ref, o_ref):
    h = e_ref[...] * hs_ref[...] + p0_ref[...] + p1_ref[...]
    t = jnp.dot(h, w_ref[...], preferred_element_type=jnp.float32) + b_ref[...]
    o_ref[...] = _maxk_mask(t)


def _tc_out_body(hs_ref, p0_ref, p1_ref, e_ref, w_ref, b_ref, o_ref):
    h = e_ref[...] * hs_ref[...] + p0_ref[...] + p1_ref[...]
    o_ref[...] = jnp.dot(h, w_ref[...], preferred_element_type=jnp.float32) + b_ref[...]


def _rows():
    return pl.BlockSpec((BR, D), lambda i: (i, 0))


def _mat():
    return pl.BlockSpec((D, D), lambda i: (0, 0))


def _vec():
    return pl.BlockSpec((1, D), lambda i: (0, 0))


_out_rows = jax.ShapeDtypeStruct((N, D), jnp.float32)

_tc_in = pl.pallas_call(
    _tc_in_body, grid=(N // BR,),
    in_specs=[_rows(), _mat(), _vec(), _mat(), _vec()],
    out_specs=_rows(), out_shape=_out_rows)

_tc_mid = pl.pallas_call(
    _tc_mid_body, grid=(N // BR,),
    in_specs=[_rows(), _rows(), _rows(), _vec(), _mat(), _vec()],
    out_specs=_rows(), out_shape=_out_rows)

_tc_out = pl.pallas_call(
    _tc_out_body, grid=(N // BR,),
    in_specs=[_rows(), _rows(), _rows(), _vec(), _mat(), _vec()],
    out_specs=_rows(), out_shape=_out_rows)


@functools.lru_cache(maxsize=None)
def _make_sc_agg():
    mesh = plsc.VectorSubcoreMesh(core_axis_name="c", subcore_axis_name="s")
    return pl.kernel(
        _sc_agg_body,
        mesh=mesh,
        out_type=jax.ShapeDtypeStruct((NC * N, D), jnp.float32),
        scratch_types=[pltpu.VMEM((CHUNK, D), jnp.float32) for _ in range(NRING)]
        + [pltpu.VMEM((CHUNK,), jnp.int32) for _ in range(2 * NRING)]
        + [pltpu.VMEM((ZR, D), jnp.float32)]
        + [pltpu.VMEM_SHARED((N, D), jnp.float32)]
        + [pltpu.SemaphoreType.DMA for _ in range(4 * NRING)],
    )


def _sc_agg_body(hs_hbm, src_hbm, dst_hbm, out_hbm, *rest):
    rows = rest[0:NRING]
    sbuf = rest[NRING:2 * NRING]
    dbuf = rest[2 * NRING:3 * NRING]
    zbuf = rest[3 * NRING]
    acc_sh = rest[3 * NRING + 1]
    sems = rest[3 * NRING + 2:]
    gs = sems[0:NRING]
    ss = sems[NRING:2 * NRING]
    isx = sems[2 * NRING:3 * NRING]
    jsx = sems[3 * NRING:4 * NRING]

    cid = lax.axis_index("c")
    sid = lax.axis_index("s")
    wg = cid * NS + sid

    def _zrow(r, carry):
        for c in range(D // 16):
            zbuf[r, pl.ds(c * 16, 16)] = jnp.zeros((16,), jnp.float32)
        return carry

    lax.fori_loop(0, ZR, _zrow, 0)

    base = pl.multiple_of(sid * RPW, 16)

    def _zcp(j, carry):
        pltpu.sync_copy(zbuf, acc_sh.at[pl.ds(base + j * ZR, ZR)])
        return carry

    lax.fori_loop(0, RPW // ZR, _zcp, 0)

    @pl.when(sid == NS - 1)
    def _zero_tail():
        pltpu.sync_copy(zbuf, acc_sh.at[pl.ds(NS * RPW, N - NS * RPW)])

    plsc.subcore_barrier()

    def fire_isrc(c, b):
        pltpu.async_copy(src_hbm.at[wg, c], sbuf[b], isx[b])

    def wait_isrc(b):
        pltpu.make_async_copy(src_hbm.at[wg, 0], sbuf[b], isx[b]).wait()

    def fire_idst(c, b):
        pltpu.async_copy(dst_hbm.at[wg, c], dbuf[b], jsx[b])

    def wait_idst(b):
        pltpu.make_async_copy(dst_hbm.at[wg, 0], dbuf[b], jsx[b]).wait()

    def fire_gather(b):
        pltpu.async_copy(hs_hbm.at[sbuf[b]], rows[b], gs[b])

    def wait_gather(b):
        pltpu.make_async_copy(hs_hbm.at[sbuf[b]], rows[b], gs[b]).wait()

    def fire_scatter(b):
        pltpu.async_copy(rows[b], acc_sh.at[dbuf[b]], ss[b], add=True)

    def wait_scatter(b):
        pltpu.make_async_copy(rows[b], acc_sh.at[dbuf[b]], ss[b]).wait()

    for b in range(NRING):
        fire_isrc(b, b)
        fire_idst(b, b)
    for b in range(NRING):
        wait_isrc(b)
        fire_gather(b)

    def _grp(k, carry):
        for b in range(NRING):
            c = k * NRING + b
            wait_gather(b)
            wait_idst(b)
            fire_scatter(b)

            @pl.when(c + NRING < NCHUNK)
            def _pre(b=b, c=c):
                fire_isrc(c + NRING, b)

            wait_scatter(b)

            @pl.when(c + NRING < NCHUNK)
            def _nxt(b=b, c=c):
                fire_idst(c + NRING, b)
                wait_isrc(b)
                fire_gather(b)
        return carry

    lax.fori_loop(0, NCHUNK // NRING, _grp, 0)

    for b in range(NTAIL):
        wait_gather(b)
        wait_idst(b)
        fire_scatter(b)
        wait_scatter(b)

    plsc.subcore_barrier()
    obase = pl.multiple_of(cid * N + sid * RPW, 16)
    pltpu.sync_copy(acc_sh.at[pl.ds(base, RPW)], out_hbm.at[pl.ds(obase, RPW)])

    @pl.when(sid == NS - 1)
    def _write_tail():
        pltpu.sync_copy(acc_sh.at[pl.ds(NS * RPW, N - NS * RPW)],
                        out_hbm.at[pl.ds(cid * N + NS * RPW, N - NS * RPW)])


def kernel(x, edge_index, W_in, b_in, Ws, bs, eps, W_out, b_out):
    src = edge_index[0].astype(jnp.int32).reshape(NW, NCHUNK, CHUNK)
    dst = edge_index[1].astype(jnp.int32).reshape(NW, NCHUNK, CHUNK)
    ones = jnp.ones((1, D), jnp.float32)

    hs = _tc_in(x, W_in, b_in.reshape(1, D), Ws[0], bs[0].reshape(1, D))
    for i in range(L):
        part = _make_sc_agg()(hs, src, dst)
        p0, p1 = part[:N], part[N:]
        epsb = (1.0 + eps[i]) * ones
        if i < L - 1:
            hs = _tc_mid(hs, p0, p1, epsb, Ws[i + 1], bs[i + 1].reshape(1, D))
        else:
            return _tc_out(hs, p0, p1, epsb, W_out, b_out.reshape(1, D))


# speedup vs baseline: 2.4501x; 1.0121x over previous
import functools

import jax
import jax.numpy as jnp
from jax import lax
from jax.experimental import pallas as pl
from jax.experimental.pallas import tpu as pltpu
from jax.experimental.pallas import tpu_sc as plsc

N = 10000
E = 320000
D = 128
L = 3
K = 32

NC = 2
NS = 16
NW = NC * NS
EPW = E // NW
CHUNK = 80
NCHUNK = EPW // CHUNK
NRING = 4
NTAIL = NCHUNK - (NCHUNK // NRING) * NRING
RPW = 624
ZR = 16

BR = 2000


def _maxk_mask(t, _roll=None):
    if _roll is None:
        _roll = lambda v, s: pltpu.roll(v, s, 1)
    n = t.shape[1]
    cols = lax.broadcasted_iota(jnp.int32, t.shape, 1)
    u = lax.bitcast_convert_type(t, jnp.int32)
    m = u ^ (lax.shift_right_arithmetic(u, 31) & jnp.int32(0x7FFFFFFF))
    key0 = (m & jnp.int32(-128)) | (127 - cols)
    x = key0
    k = 2
    while k <= n // 2:
        kbit0 = (cols & k) == 0
        j = k // 2
        while j >= 1:
            jbit0 = (cols & j) == 0
            pv = jnp.where(jbit0, _roll(x, n - j), _roll(x, j))
            take_min = kbit0 == jbit0
            x = jnp.where(take_min, jnp.minimum(x, pv), jnp.maximum(x, pv))
            j //= 2
        k *= 2
    for j in (n // 2, n // 4):
        jbit0 = (cols & j) == 0
        pv = jnp.where(jbit0, _roll(x, n - j), _roll(x, j))
        x = jnp.where(jbit0, jnp.minimum(x, pv), jnp.maximum(x, pv))
    m = jnp.where(cols >= n - K, x, jnp.int32(0x7FFFFFFF))
    sh = K // 2
    while sh >= 1:
        m = jnp.minimum(m, _roll(m, n - sh))
        sh //= 2
    thr = lax.slice(m, (0, n - K), (t.shape[0], n - K + 1))
    return jnp.where(key0 >= thr, t, 0.0)


def _tc_in_body(x_ref, wi_ref, bi_ref, w0_ref, b0_ref, o_ref):
    h0 = jnp.dot(x_ref[...], wi_ref[...], preferred_element_type=jnp.float32)
    h0 = jnp.maximum(h0 + bi_ref[...], 0.0)
    t = jnp.dot(h0, w0_ref[...], preferred_element_type=jnp.float32) + b0_ref[...]
    o_ref[...] = _maxk_mask(t)


def _tc_mid_body(hs_ref, p0_ref, p1_ref, e_ref, w_ref, b_ref, o_ref):
    h = e_ref[...] * hs_ref[...] + p0_ref[...] + p1_ref[...]
    t = jnp.dot(h, w_ref[...], preferred_element_type=jnp.float32) + b_ref[...]
    o_ref[...] = _maxk_mask(t)


def _tc_out_body(hs_ref, p0_ref, p1_ref, e_ref, w_ref, b_ref, o_ref):
    h = e_ref[...] * hs_ref[...] + p0_ref[...] + p1_ref[...]
    o_ref[...] = jnp.dot(h, w_ref[...], preferred_element_type=jnp.float32) + b_ref[...]


def _rows():
    return pl.BlockSpec((BR, D), lambda i: (i, 0))


def _mat():
    return pl.BlockSpec((D, D), lambda i: (0, 0))


def _vec():
    return pl.BlockSpec((1, D), lambda i: (0, 0))


_out_rows = jax.ShapeDtypeStruct((N, D), jnp.float32)

_tc_in = pl.pallas_call(
    _tc_in_body, grid=(N // BR,),
    in_specs=[_rows(), _mat(), _vec(), _mat(), _vec()],
    out_specs=_rows(), out_shape=_out_rows)

_tc_mid = pl.pallas_call(
    _tc_mid_body, grid=(N // BR,),
    in_specs=[_rows(), _rows(), _rows(), _vec(), _mat(), _vec()],
    out_specs=_rows(), out_shape=_out_rows)

_tc_out = pl.pallas_call(
    _tc_out_body, grid=(N // BR,),
    in_specs=[_rows(), _rows(), _rows(), _vec(), _mat(), _vec()],
    out_specs=_rows(), out_shape=_out_rows)


@functools.lru_cache(maxsize=None)
def _make_sc_agg():
    mesh = plsc.VectorSubcoreMesh(core_axis_name="c", subcore_axis_name="s")
    return pl.kernel(
        _sc_agg_body,
        mesh=mesh,
        out_type=jax.ShapeDtypeStruct((NC * N, D), jnp.float32),
        scratch_types=[pltpu.VMEM((CHUNK, D), jnp.float32) for _ in range(NRING)]
        + [pltpu.VMEM((CHUNK,), jnp.int32) for _ in range(2 * NRING)]
        + [pltpu.VMEM((ZR, D), jnp.float32)]
        + [pltpu.VMEM_SHARED((N, D), jnp.float32)]
        + [pltpu.SemaphoreType.DMA for _ in range(4 * NRING)],
    )


def _sc_agg_body(hs_hbm, src_hbm, dst_hbm, out_hbm, *rest):
    rows = rest[0:NRING]
    sbuf = rest[NRING:2 * NRING]
    dbuf = rest[2 * NRING:3 * NRING]
    zbuf = rest[3 * NRING]
    acc_sh = rest[3 * NRING + 1]
    sems = rest[3 * NRING + 2:]
    gs = sems[0:NRING]
    ss = sems[NRING:2 * NRING]
    isx = sems[2 * NRING:3 * NRING]
    jsx = sems[3 * NRING:4 * NRING]

    cid = lax.axis_index("c")
    sid = lax.axis_index("s")
    wg = cid * NS + sid

    def fire_isrc(c, b):
        pltpu.async_copy(src_hbm.at[wg, c], sbuf[b], isx[b])

    def wait_isrc(b):
        pltpu.make_async_copy(src_hbm.at[wg, 0], sbuf[b], isx[b]).wait()

    def fire_idst(c, b):
        pltpu.async_copy(dst_hbm.at[wg, c], dbuf[b], jsx[b])

    def wait_idst(b):
        pltpu.make_async_copy(dst_hbm.at[wg, 0], dbuf[b], jsx[b]).wait()

    def fire_gather(b):
        pltpu.async_copy(hs_hbm.at[sbuf[b]], rows[b], gs[b])

    def wait_gather(b):
        pltpu.make_async_copy(hs_hbm.at[sbuf[b]], rows[b], gs[b]).wait()

    def fire_scatter(b):
        pltpu.async_copy(rows[b], acc_sh.at[dbuf[b]], ss[b], add=True)

    def wait_scatter(b):
        pltpu.make_async_copy(rows[b], acc_sh.at[dbuf[b]], ss[b]).wait()

    for b in range(NRING):
        fire_isrc(b, b)
        fire_idst(b, b)
    for b in range(NRING):
        wait_isrc(b)
        fire_gather(b)

    def _zrow(r, carry):
        for c in range(D // 16):
            zbuf[r, pl.ds(c * 16, 16)] = jnp.zeros((16,), jnp.float32)
        return carry

    lax.fori_loop(0, ZR, _zrow, 0)

    base = pl.multiple_of(sid * RPW, 16)

    def _zcp(j, carry):
        pltpu.sync_copy(zbuf, acc_sh.at[pl.ds(base + j * ZR, ZR)])
        return carry

    lax.fori_loop(0, RPW // ZR, _zcp, 0)

    @pl.when(sid == NS - 1)
    def _zero_tail():
        pltpu.sync_copy(zbuf, acc_sh.at[pl.ds(NS * RPW, N - NS * RPW)])

    plsc.subcore_barrier()

    def _grp(k, carry):
        for b in range(NRING):
            c = k * NRING + b
            wait_gather(b)
            wait_idst(b)
            fire_scatter(b)

            @pl.when(c + NRING < NCHUNK)
            def _pre(b=b, c=c):
                fire_isrc(c + NRING, b)

            wait_scatter(b)

            @pl.when(c + NRING < NCHUNK)
            def _nxt(b=b, c=c):
                fire_idst(c + NRING, b)
                wait_isrc(b)
                fire_gather(b)
        return carry

    lax.fori_loop(0, NCHUNK // NRING, _grp, 0)

    for b in range(NTAIL):
        wait_gather(b)
        wait_idst(b)
        fire_scatter(b)
        wait_scatter(b)

    plsc.subcore_barrier()
    obase = pl.multiple_of(cid * N + sid * RPW, 16)
    pltpu.sync_copy(acc_sh.at[pl.ds(base, RPW)], out_hbm.at[pl.ds(obase, RPW)])

    @pl.when(sid == NS - 1)
    def _write_tail():
        pltpu.sync_copy(acc_sh.at[pl.ds(NS * RPW, N - NS * RPW)],
                        out_hbm.at[pl.ds(cid * N + NS * RPW, N - NS * RPW)])


def kernel(x, edge_index, W_in, b_in, Ws, bs, eps, W_out, b_out):
    src = edge_index[0].astype(jnp.int32).reshape(NW, NCHUNK, CHUNK)
    dst = edge_index[1].astype(jnp.int32).reshape(NW, NCHUNK, CHUNK)
    ones = jnp.ones((1, D), jnp.float32)

    hs = _tc_in(x, W_in, b_in.reshape(1, D), Ws[0], bs[0].reshape(1, D))
    for i in range(L):
        part = _make_sc_agg()(hs, src, dst)
        p0, p1 = part[:N], part[N:]
        epsb = (1.0 + eps[i]) * ones
        if i < L - 1:
            hs = _tc_mid(hs, p0, p1, epsb, Ws[i + 1], bs[i + 1].reshape(1, D))
        else:
            return _tc_out(hs, p0, p1, epsb, W_out, b_out.reshape(1, D))
